# baseline (speedup 1.0000x reference)
import functools

import jax
import jax.numpy as jnp
from jax import lax
from jax.experimental import pallas as pl
from jax.experimental.pallas import tpu as pltpu
from jax.experimental.pallas import tpu_sc as plsc

N = 10000
E = 320000
D = 128
NLANE = 16

K = 64
NCH = E // K
NTILES = 32
RPT = 624
TAIL = N - 16 * RPT
DW = D + NLANE


def _lane_bcast(v, j):
    idx = jnp.full((NLANE, 1), j, dtype=jnp.int32)
    return lax.gather(
        v, idx,
        lax.GatherDimensionNumbers(offset_dims=(), collapsed_slice_dims=(0,),
                                   start_index_map=(0,)),
        (1,), mode=lax.GatherScatterMode.PROMISE_IN_BOUNDS)


def _make_edge_kernel(heads):
    mesh = plsc.VectorSubcoreMesh(core_axis_name="c", subcore_axis_name="s")

    @functools.partial(
        pl.kernel,
        out_type=(
            jax.ShapeDtypeStruct((2, N, D), jnp.float32),
            jax.ShapeDtypeStruct((2, N, NLANE), jnp.float32),
        ),
        mesh=mesh,
        compiler_params=pltpu.CompilerParams(use_tc_tiling_on_sc=False),
        scratch_types=[
            pltpu.VMEM((2, 2, K), jnp.int32),
            pltpu.VMEM((2, K), jnp.int32),
            pltpu.VMEM((2, K, NLANE), jnp.float32),
            pltpu.VMEM((2, K, NLANE), jnp.float32),
            pltpu.VMEM((2, K, D), jnp.float32),
            pltpu.VMEM((2, K, D), jnp.float32),
            pltpu.VMEM((2, K, NLANE), jnp.float32),
            pltpu.VMEM_SHARED((N, D), jnp.float32),
            pltpu.VMEM_SHARED((N, NLANE), jnp.float32),
            [pltpu.SemaphoreType.DMA] * 12,
        ],
    )
    def edge_kernel(esd_h, h_h, asrc_h, adst_h, z_h, z16_h, out_h, den_h,
                    eidx, sidx, ars, ard, hrows, msg, wbuf,
                    acc, dacc, sems):
        c = lax.axis_index("c")
        s = lax.axis_index("s")
        wid = s * 2 + c
        base = s * RPT
        si = [sems[0], sems[1]]
        sga = [sems[2], sems[3]]
        sgd = [sems[4], sems[5]]
        sgh = [sems[6], sems[7]]
        ssw = [sems[8], sems[9]]
        ssm = [sems[10], sems[11]]

        pltpu.sync_copy(z_h.at[pl.ds(base, RPT)], acc.at[pl.ds(base, RPT)])
        pltpu.sync_copy(z16_h.at[pl.ds(base, RPT)], dacc.at[pl.ds(base, RPT)])

        @pl.when(s == 15)
        def _():
            pltpu.sync_copy(z_h.at[pl.ds(16 * RPT, TAIL)],
                            acc.at[pl.ds(16 * RPT, TAIL)])
            pltpu.sync_copy(z16_h.at[pl.ds(16 * RPT, TAIL)],
                            dacc.at[pl.ds(16 * RPT, TAIL)])
        plsc.subcore_barrier()

        nch = (NCH - wid + NTILES - 1) // NTILES

        def cid(k):
            return wid + k * NTILES

        def gathers_start(p, b):
            pltpu.async_copy(asrc_h.at[eidx.at[b, 0]], ars.at[b], sga[p])
            pltpu.async_copy(adst_h.at[eidx.at[b, 1]], ard.at[b], sgd[p])
            pltpu.async_copy(h_h.at[eidx.at[b, 0]], hrows.at[b], sgh[p])

        def gathers_wait(p, b):
            pltpu.make_async_copy(asrc_h.at[eidx.at[b, 0]], ars.at[b],
                                  sga[p]).wait()
            pltpu.make_async_copy(adst_h.at[eidx.at[b, 1]], ard.at[b],
                                  sgd[p]).wait()
            pltpu.make_async_copy(h_h.at[eidx.at[b, 0]], hrows.at[b],
                                  sgh[p]).wait()

        def scatters_wait(p, b):
            pltpu.make_async_copy(wbuf.at[b], dacc.at[sidx.at[b]],
                                  ssw[p]).wait()
            pltpu.make_async_copy(msg.at[b], acc.at[sidx.at[b]],
                                  ssm[p]).wait()

        def phase(k, p):
            @pl.when(k + 1 < nch)
            def _():
                q = 1 - p
                pltpu.make_async_copy(esd_h.at[cid(k + 1)], eidx.at[q],
                                      si[q]).wait()
                gathers_start(q, q)

            gathers_wait(p, p)

            @pl.when(k + 2 < nch)
            def _():
                pltpu.async_copy(esd_h.at[cid(k + 2)], eidx.at[p], si[p])

            @pl.when(k >= 2)
            def _():
                scatters_wait(p, p)

            for t in range(K // NLANE):
                sidx[p, pl.ds(NLANE * t, NLANE)] = \
                    eidx[p, 1, pl.ds(NLANE * t, NLANE)]

            @plsc.parallel_loop(0, K, unroll=8)
            def _edge(i):
                a = ars[p, i, :] + ard[p, i, :]
                w16 = jnp.exp(jnp.maximum(a, a * 0.2))
                wbuf[p, i, :] = w16
                for j in range(D // NLANE):
                    hij = hrows[p, i, pl.ds(NLANE * j, NLANE)]
                    wj = w16 if heads == 1 else _lane_bcast(w16, j)
                    msg[p, i, pl.ds(NLANE * j, NLANE)] = hij * wj

            pltpu.async_copy(wbuf.at[p], dacc.at[sidx.at[p]], ssw[p],
                             add=True)
            pltpu.async_copy(msg.at[p], acc.at[sidx.at[p]], ssm[p],
                             add=True)

        pltpu.sync_copy(esd_h.at[cid(0)], eidx.at[0])
        pltpu.async_copy(esd_h.at[cid(1)], eidx.at[1], si[1])
        gathers_start(0, 0)

        def pair(t, carry):
            phase(2 * t, 0)
            phase(2 * t + 1, 1)
            return carry
        lax.fori_loop(0, nch // 2, pair, 0)

        @pl.when(nch % 2 == 1)
        def _():
            phase(nch - 1, 0)

        scatters_wait(0, 0)
        scatters_wait(1, 1)

        plsc.subcore_barrier()
        pltpu.sync_copy(acc.at[pl.ds(base, RPT)], out_h.at[c, pl.ds(base, RPT)])
        pltpu.sync_copy(dacc.at[pl.ds(base, RPT)], den_h.at[c, pl.ds(base, RPT)])

        @pl.when(s == 15)
        def _():
            pltpu.sync_copy(acc.at[pl.ds(16 * RPT, TAIL)],
                            out_h.at[c, pl.ds(16 * RPT, TAIL)])
            pltpu.sync_copy(dacc.at[pl.ds(16 * RPT, TAIL)],
                            den_h.at[c, pl.ds(16 * RPT, TAIL)])

    return edge_kernel


_edge8 = _make_edge_kernel(8)
_edge1 = _make_edge_kernel(1)


BN = 1000


def _dense0_body(x_ref, w_ref, as_ref, ad_ref, h_ref, asrc_ref, adst_ref, ws_ref):
    xb = x_ref[...]
    h = jnp.dot(xb, w_ref[...], preferred_element_type=jnp.float32)
    h_ref[...] = h
    a_s = jnp.dot(h, as_ref[...], preferred_element_type=jnp.float32)
    a_d = jnp.dot(h, ad_ref[...], preferred_element_type=jnp.float32)
    asrc_ref[...] = a_s
    adst_ref[...] = a_d
    z = a_s + a_d
    ws_ref[...] = jnp.exp(jnp.maximum(z, 0.2 * z))


def _comb0_body(p0_ref, p1_ref, d0_ref, d1_ref, ws_ref, h0_ref, b0_ref, r_ref,
                w1_ref, a1s_ref, a1d_ref, h1m_ref, as1_ref, ad1_ref, ws1_ref):
    ws = ws_ref[...]
    den = d0_ref[...] + d1_ref[...] + ws
    inv = 1.0 / (den + 1e-16)
    r = r_ref[...]
    acc = p0_ref[...] + p1_ref[...] + h0_ref[...] * jnp.dot(
        ws, r, preferred_element_type=jnp.float32)
    out0 = acc * jnp.dot(inv, r, preferred_element_type=jnp.float32) + b0_ref[...]
    h1 = jnp.where(out0 > 0, out0, jnp.exp(out0) - 1.0)
    h1m = jnp.dot(h1, w1_ref[...], preferred_element_type=jnp.float32)
    h1m_ref[...] = h1m
    a1s = jnp.dot(h1m, a1s_ref[...], preferred_element_type=jnp.float32)
    a1d = jnp.dot(h1m, a1d_ref[...], preferred_element_type=jnp.float32)
    as1_ref[...] = a1s
    ad1_ref[...] = a1d
    z = a1s + a1d
    ws1_ref[...] = jnp.exp(jnp.maximum(z, 0.2 * z))


def _final_body(q0_ref, q1_ref, e0_ref, e1_ref, ws1_ref, h1m_ref, b1_ref,
                r1_ref, wp_ref, bp_ref, emb_ref, g_ref, gacc):
    i = pl.program_id(0)
    ws1 = ws1_ref[...]
    den = e0_ref[...] + e1_ref[...] + ws1
    inv = 1.0 / (den + 1e-16)
    r1 = r1_ref[...]
    acc = q0_ref[...] + q1_ref[...] + h1m_ref[...] * jnp.dot(
        ws1, r1, preferred_element_type=jnp.float32)
    emb = acc * jnp.dot(inv, r1, preferred_element_type=jnp.float32) + b1_ref[...]
    emb_ref[...] = emb
    part = jnp.sum(emb, axis=0, keepdims=True)

    @pl.when(i == 0)
    def _():
        gacc[...] = part

    @pl.when(i > 0)
    def _():
        gacc[...] = gacc[...] + part

    @pl.when(i == (N // BN) - 1)
    def _():
        g = gacc[...] * (1.0 / N)
        g_ref[...] = jnp.maximum(
            jnp.dot(g, wp_ref[...], preferred_element_type=jnp.float32)
            + bp_ref[...], 0.0)


def _blk(shape, imap):
    return pl.BlockSpec(shape, imap)


def kernel(x, edge_index, W0, att_src0, att_dst0, b0, W1, att_src1, att_dst1,
           b1, Wp, bp):
    f32 = jnp.float32
    ei = edge_index.astype(jnp.int32)
    esd = ei.reshape(2, NCH, K).transpose(1, 0, 2)

    mask = (jnp.arange(D)[:, None] // NLANE
            == jnp.arange(NLANE)[None, :]).astype(f32)
    As0 = att_src0.reshape(D, 1) * mask
    Ad0 = att_dst0.reshape(D, 1) * mask
    R = mask.T
    At1s = att_src1.reshape(D, 1) * jnp.ones((1, NLANE), f32)
    At1d = att_dst1.reshape(D, 1) * jnp.ones((1, NLANE), f32)
    R1 = (jnp.arange(NLANE)[:, None] == 0).astype(f32) * jnp.ones((1, D), f32)
    zeros = jnp.zeros((N, D), f32)
    zeros16 = jnp.zeros((N, NLANE), f32)
    b0r = b0.reshape(1, D)
    b1r = b1.reshape(1, D)
    bpr = bp.reshape(1, D)

    grid = (N // BN,)
    row = lambda i: (i, 0)
    fix = lambda i: (0, 0)

    h0, asrc0, adst0, ws0 = pl.pallas_call(
        _dense0_body,
        grid=grid,
        in_specs=[_blk((BN, D), row), _blk((D, D), fix),
                  _blk((D, NLANE), fix), _blk((D, NLANE), fix)],
        out_specs=[_blk((BN, D), row), _blk((BN, NLANE), row),
                   _blk((BN, NLANE), row), _blk((BN, NLANE), row)],
        out_shape=[jax.ShapeDtypeStruct((N, D), f32),
                   jax.ShapeDtypeStruct((N, NLANE), f32),
                   jax.ShapeDtypeStruct((N, NLANE), f32),
                   jax.ShapeDtypeStruct((N, NLANE), f32)],
    )(x, W0, As0, Ad0)

    outp, denp = _edge8(esd, h0, asrc0, adst0, zeros, zeros16)

    h1m, as1, ad1, ws1 = pl.pallas_call(
        _comb0_body,
        grid=grid,
        in_specs=[_blk((BN, D), row), _blk((BN, D), row),
                  _blk((BN, NLANE), row), _blk((BN, NLANE), row),
                  _blk((BN, NLANE), row), _blk((BN, D), row),
                  _blk((1, D), fix), _blk((NLANE, D), fix),
                  _blk((D, D), fix), _blk((D, NLANE), fix),
                  _blk((D, NLANE), fix)],
        out_specs=[_blk((BN, D), row), _blk((BN, NLANE), row),
                   _blk((BN, NLANE), row), _blk((BN, NLANE), row)],
        out_shape=[jax.ShapeDtypeStruct((N, D), f32),
                   jax.ShapeDtypeStruct((N, NLANE), f32),
                   jax.ShapeDtypeStruct((N, NLANE), f32),
                   jax.ShapeDtypeStruct((N, NLANE), f32)],
    )(outp[0], outp[1], denp[0], denp[1], ws0, h0, b0r, R, W1, At1s, At1d)

    outp1, denp1 = _edge1(esd, h1m, as1, ad1, zeros, zeros16)

    emb, g = pl.pallas_call(
        _final_body,
        grid=grid,
        in_specs=[_blk((BN, D), row), _blk((BN, D), row),
                  _blk((BN, NLANE), row), _blk((BN, NLANE), row),
                  _blk((BN, NLANE), row), _blk((BN, D), row),
                  _blk((1, D), fix), _blk((NLANE, D), fix),
                  _blk((D, D), fix), _blk((1, D), fix)],
        out_specs=[_blk((BN, D), row), _blk((1, D), fix)],
        out_shape=[jax.ShapeDtypeStruct((N, D), f32),
                   jax.ShapeDtypeStruct((1, D), f32)],
        scratch_shapes=[pltpu.VMEM((1, D), f32)],
    )(outp1[0], outp1[1], denp1[0], denp1[1], ws1, h1m, b1r, R1, Wp, bpr)

    return (emb, g)


# speedup vs baseline: 1.0176x; 1.0176x over previous
import functools

import jax
import jax.numpy as jnp
from jax import lax
from jax.experimental import pallas as pl
from jax.experimental.pallas import tpu as pltpu
from jax.experimental.pallas import tpu_sc as plsc

N = 10000
E = 320000
D = 128
NLANE = 16

K = 64
NCH = E // K
NTILES = 32
RPT = 624
TAIL = N - 16 * RPT
DW = D + NLANE


def _lane_bcast(v, j):
    idx = jnp.full((NLANE, 1), j, dtype=jnp.int32)
    return lax.gather(
        v, idx,
        lax.GatherDimensionNumbers(offset_dims=(), collapsed_slice_dims=(0,),
                                   start_index_map=(0,)),
        (1,), mode=lax.GatherScatterMode.PROMISE_IN_BOUNDS)


def _make_edge_kernel(heads):
    mesh = plsc.VectorSubcoreMesh(core_axis_name="c", subcore_axis_name="s")

    @functools.partial(
        pl.kernel,
        out_type=(
            jax.ShapeDtypeStruct((2, N, D), jnp.float32),
            jax.ShapeDtypeStruct((2, N, NLANE), jnp.float32),
        ),
        mesh=mesh,
        compiler_params=pltpu.CompilerParams(use_tc_tiling_on_sc=False),
        scratch_types=[
            pltpu.VMEM((2, 2, K), jnp.int32),
            pltpu.VMEM((2, K), jnp.int32),
            pltpu.VMEM((2, K, NLANE), jnp.float32),
            pltpu.VMEM((2, K, NLANE), jnp.float32),
            pltpu.VMEM((2, K, D), jnp.float32),
            pltpu.VMEM((2, K, D), jnp.float32),
            pltpu.VMEM((2, K, NLANE), jnp.float32),
            pltpu.VMEM_SHARED((N, D), jnp.float32),
            pltpu.VMEM_SHARED((N, NLANE), jnp.float32),
            [pltpu.SemaphoreType.DMA] * 12,
        ],
    )
    def edge_kernel(esd_h, h_h, asrc_h, adst_h, z_h, z16_h, out_h, den_h,
                    eidx, sidx, ars, ard, hrows, msg, wbuf,
                    acc, dacc, sems):
        c = lax.axis_index("c")
        s = lax.axis_index("s")
        wid = s * 2 + c
        base = s * RPT
        si = [sems[0], sems[1]]
        sga = [sems[2], sems[3]]
        sgd = [sems[4], sems[5]]
        sgh = [sems[6], sems[7]]
        ssw = [sems[8], sems[9]]
        ssm = [sems[10], sems[11]]

        pltpu.sync_copy(z_h.at[pl.ds(base, RPT)], acc.at[pl.ds(base, RPT)])
        pltpu.sync_copy(z16_h.at[pl.ds(base, RPT)], dacc.at[pl.ds(base, RPT)])

        @pl.when(s == 15)
        def _():
            pltpu.sync_copy(z_h.at[pl.ds(16 * RPT, TAIL)],
                            acc.at[pl.ds(16 * RPT, TAIL)])
            pltpu.sync_copy(z16_h.at[pl.ds(16 * RPT, TAIL)],
                            dacc.at[pl.ds(16 * RPT, TAIL)])
        plsc.subcore_barrier()

        nch = (NCH - wid + NTILES - 1) // NTILES

        def cid(k):
            return wid + k * NTILES

        def gathers_start(p, b):
            pltpu.async_copy(asrc_h.at[eidx.at[b, 0]], ars.at[b], sga[p])
            pltpu.async_copy(adst_h.at[eidx.at[b, 1]], ard.at[b], sgd[p])
            pltpu.async_copy(h_h.at[eidx.at[b, 0]], hrows.at[b], sgh[p])

        def gathers_wait(p, b):
            pltpu.make_async_copy(asrc_h.at[eidx.at[b, 0]], ars.at[b],
                                  sga[p]).wait()
            pltpu.make_async_copy(adst_h.at[eidx.at[b, 1]], ard.at[b],
                                  sgd[p]).wait()
            pltpu.make_async_copy(h_h.at[eidx.at[b, 0]], hrows.at[b],
                                  sgh[p]).wait()

        def scatters_wait(p, b):
            pltpu.make_async_copy(wbuf.at[b], dacc.at[sidx.at[b]],
                                  ssw[p]).wait()
            pltpu.make_async_copy(msg.at[b], acc.at[sidx.at[b]],
                                  ssm[p]).wait()

        def phase(k, p):
            @pl.when(k + 1 < nch)
            def _():
                q = 1 - p
                pltpu.make_async_copy(esd_h.at[cid(k + 1)], eidx.at[q],
                                      si[q]).wait()
                gathers_start(q, q)

            gathers_wait(p, p)

            @pl.when(k + 2 < nch)
            def _():
                pltpu.async_copy(esd_h.at[cid(k + 2)], eidx.at[p], si[p])

            @pl.when(k >= 2)
            def _():
                scatters_wait(p, p)

            for t in range(K // NLANE):
                sidx[p, pl.ds(NLANE * t, NLANE)] = \
                    eidx[p, 1, pl.ds(NLANE * t, NLANE)]

            @plsc.parallel_loop(0, K, unroll=4)
            def _edge(i):
                a = ars[p, i, :] + ard[p, i, :]
                w16 = jnp.exp(jnp.maximum(a, a * 0.2))
                wbuf[p, i, :] = w16
                for j in range(D // NLANE):
                    hij = hrows[p, i, pl.ds(NLANE * j, NLANE)]
                    wj = w16 if heads == 1 else _lane_bcast(w16, j)
                    msg[p, i, pl.ds(NLANE * j, NLANE)] = hij * wj

            pltpu.async_copy(wbuf.at[p], dacc.at[sidx.at[p]], ssw[p],
                             add=True)
            pltpu.async_copy(msg.at[p], acc.at[sidx.at[p]], ssm[p],
                             add=True)

        pltpu.sync_copy(esd_h.at[cid(0)], eidx.at[0])
        pltpu.async_copy(esd_h.at[cid(1)], eidx.at[1], si[1])
        gathers_start(0, 0)

        def pair(t, carry):
            phase(2 * t, 0)
            phase(2 * t + 1, 1)
            return carry
        lax.fori_loop(0, nch // 2, pair, 0)

        @pl.when(nch % 2 == 1)
        def _():
            phase(nch - 1, 0)

        scatters_wait(0, 0)
        scatters_wait(1, 1)

        plsc.subcore_barrier()
        pltpu.sync_copy(acc.at[pl.ds(base, RPT)], out_h.at[c, pl.ds(base, RPT)])
        pltpu.sync_copy(dacc.at[pl.ds(base, RPT)], den_h.at[c, pl.ds(base, RPT)])

        @pl.when(s == 15)
        def _():
            pltpu.sync_copy(acc.at[pl.ds(16 * RPT, TAIL)],
                            out_h.at[c, pl.ds(16 * RPT, TAIL)])
            pltpu.sync_copy(dacc.at[pl.ds(16 * RPT, TAIL)],
                            den_h.at[c, pl.ds(16 * RPT, TAIL)])

    return edge_kernel


_edge8 = _make_edge_kernel(8)
_edge1 = _make_edge_kernel(1)


BN = 1000


def _dense0_body(x_ref, w_ref, as_ref, ad_ref, h_ref, asrc_ref, adst_ref, ws_ref):
    xb = x_ref[...]
    h = jnp.dot(xb, w_ref[...], preferred_element_type=jnp.float32)
    h_ref[...] = h
    a_s = jnp.dot(h, as_ref[...], preferred_element_type=jnp.float32)
    a_d = jnp.dot(h, ad_ref[...], preferred_element_type=jnp.float32)
    asrc_ref[...] = a_s
    adst_ref[...] = a_d
    z = a_s + a_d
    ws_ref[...] = jnp.exp(jnp.maximum(z, 0.2 * z))


def _comb0_body(p0_ref, p1_ref, d0_ref, d1_ref, ws_ref, h0_ref, b0_ref, r_ref,
                w1_ref, a1s_ref, a1d_ref, h1m_ref, as1_ref, ad1_ref, ws1_ref):
    ws = ws_ref[...]
    den = d0_ref[...] + d1_ref[...] + ws
    inv = 1.0 / (den + 1e-16)
    r = r_ref[...]
    acc = p0_ref[...] + p1_ref[...] + h0_ref[...] * jnp.dot(
        ws, r, preferred_element_type=jnp.float32)
    out0 = acc * jnp.dot(inv, r, preferred_element_type=jnp.float32) + b0_ref[...]
    h1 = jnp.where(out0 > 0, out0, jnp.exp(out0) - 1.0)
    h1m = jnp.dot(h1, w1_ref[...], preferred_element_type=jnp.float32)
    h1m_ref[...] = h1m
    a1s = jnp.dot(h1m, a1s_ref[...], preferred_element_type=jnp.float32)
    a1d = jnp.dot(h1m, a1d_ref[...], preferred_element_type=jnp.float32)
    as1_ref[...] = a1s
    ad1_ref[...] = a1d
    z = a1s + a1d
    ws1_ref[...] = jnp.exp(jnp.maximum(z, 0.2 * z))


def _final_body(q0_ref, q1_ref, e0_ref, e1_ref, ws1_ref, h1m_ref, b1_ref,
                r1_ref, wp_ref, bp_ref, emb_ref, g_ref, gacc):
    i = pl.program_id(0)
    ws1 = ws1_ref[...]
    den = e0_ref[...] + e1_ref[...] + ws1
    inv = 1.0 / (den + 1e-16)
    r1 = r1_ref[...]
    acc = q0_ref[...] + q1_ref[...] + h1m_ref[...] * jnp.dot(
        ws1, r1, preferred_element_type=jnp.float32)
    emb = acc * jnp.dot(inv, r1, preferred_element_type=jnp.float32) + b1_ref[...]
    emb_ref[...] = emb
    part = jnp.sum(emb, axis=0, keepdims=True)

    @pl.when(i == 0)
    def _():
        gacc[...] = part

    @pl.when(i > 0)
    def _():
        gacc[...] = gacc[...] + part

    @pl.when(i == (N // BN) - 1)
    def _():
        g = gacc[...] * (1.0 / N)
        g_ref[...] = jnp.maximum(
            jnp.dot(g, wp_ref[...], preferred_element_type=jnp.float32)
            + bp_ref[...], 0.0)


def _blk(shape, imap):
    return pl.BlockSpec(shape, imap)


def kernel(x, edge_index, W0, att_src0, att_dst0, b0, W1, att_src1, att_dst1,
           b1, Wp, bp):
    f32 = jnp.float32
    ei = edge_index.astype(jnp.int32)
    esd = ei.reshape(2, NCH, K).transpose(1, 0, 2)

    mask = (jnp.arange(D)[:, None] // NLANE
            == jnp.arange(NLANE)[None, :]).astype(f32)
    As0 = att_src0.reshape(D, 1) * mask
    Ad0 = att_dst0.reshape(D, 1) * mask
    R = mask.T
    At1s = att_src1.reshape(D, 1) * jnp.ones((1, NLANE), f32)
    At1d = att_dst1.reshape(D, 1) * jnp.ones((1, NLANE), f32)
    R1 = (jnp.arange(NLANE)[:, None] == 0).astype(f32) * jnp.ones((1, D), f32)
    zeros = jnp.zeros((N, D), f32)
    zeros16 = jnp.zeros((N, NLANE), f32)
    b0r = b0.reshape(1, D)
    b1r = b1.reshape(1, D)
    bpr = bp.reshape(1, D)

    grid = (N // BN,)
    row = lambda i: (i, 0)
    fix = lambda i: (0, 0)

    h0, asrc0, adst0, ws0 = pl.pallas_call(
        _dense0_body,
        grid=grid,
        in_specs=[_blk((BN, D), row), _blk((D, D), fix),
                  _blk((D, NLANE), fix), _blk((D, NLANE), fix)],
        out_specs=[_blk((BN, D), row), _blk((BN, NLANE), row),
                   _blk((BN, NLANE), row), _blk((BN, NLANE), row)],
        out_shape=[jax.ShapeDtypeStruct((N, D), f32),
                   jax.ShapeDtypeStruct((N, NLANE), f32),
                   jax.ShapeDtypeStruct((N, NLANE), f32),
                   jax.ShapeDtypeStruct((N, NLANE), f32)],
    )(x, W0, As0, Ad0)

    outp, denp = _edge8(esd, h0, asrc0, adst0, zeros, zeros16)

    h1m, as1, ad1, ws1 = pl.pallas_call(
        _comb0_body,
        grid=grid,
        in_specs=[_blk((BN, D), row), _blk((BN, D), row),
                  _blk((BN, NLANE), row), _blk((BN, NLANE), row),
                  _blk((BN, NLANE), row), _blk((BN, D), row),
                  _blk((1, D), fix), _blk((NLANE, D), fix),
                  _blk((D, D), fix), _blk((D, NLANE), fix),
                  _blk((D, NLANE), fix)],
        out_specs=[_blk((BN, D), row), _blk((BN, NLANE), row),
                   _blk((BN, NLANE), row), _blk((BN, NLANE), row)],
        out_shape=[jax.ShapeDtypeStruct((N, D), f32),
                   jax.ShapeDtypeStruct((N, NLANE), f32),
                   jax.ShapeDtypeStruct((N, NLANE), f32),
                   jax.ShapeDtypeStruct((N, NLANE), f32)],
    )(outp[0], outp[1], denp[0], denp[1], ws0, h0, b0r, R, W1, At1s, At1d)

    outp1, denp1 = _edge1(esd, h1m, as1, ad1, zeros, zeros16)

    emb, g = pl.pallas_call(
        _final_body,
        grid=grid,
        in_specs=[_blk((BN, D), row), _blk((BN, D), row),
                  _blk((BN, NLANE), row), _blk((BN, NLANE), row),
                  _blk((BN, NLANE), row), _blk((BN, D), row),
                  _blk((1, D), fix), _blk((NLANE, D), fix),
                  _blk((D, D), fix), _blk((1, D), fix)],
        out_specs=[_blk((BN, D), row), _blk((1, D), fix)],
        out_shape=[jax.ShapeDtypeStruct((N, D), f32),
                   jax.ShapeDtypeStruct((1, D), f32)],
        scratch_shapes=[pltpu.VMEM((1, D), f32)],
    )(outp1[0], outp1[1], denp1[0], denp1[1], ws1, h1m, b1r, R1, Wp, bpr)

    return (emb, g)
